# baseline (speedup 1.0000x reference)
import jax
import jax.numpy as jnp
from jax.experimental import pallas as pl
from jax.experimental.pallas import tpu as pltpu

N = 10000
F = 128
BM = 200
NB = N // BM
CACHE_BLOCKS = 6


def _gcn_kernel(adj_ref, x_ref, w1_ref, w2_ref, b1_ref, b2_ref,
                bmu_ref, bls_ref, mu_ref, ls_ref,
                s_ref, t_ref, cache_ref, acc_ref):
    p = pl.program_id(0)
    i = pl.program_id(1)
    in_tail = i >= NB - CACHE_BLOCKS

    @pl.when(jnp.logical_and(p == 0, i == 0))
    def _():
        s_ref[...] = jnp.dot(
            x_ref[...].astype(jnp.bfloat16), w1_ref[...],
            preferred_element_type=jnp.float32).astype(jnp.bfloat16)

    @pl.when(jnp.logical_and(p == 1, i == 0))
    def _():
        s_ref[...] = jnp.dot(
            t_ref[...], w2_ref[...],
            preferred_element_type=jnp.float32).astype(jnp.bfloat16)

    @pl.when(jnp.logical_or(p == 0, jnp.logical_not(in_tail)))
    def _():
        lhs = adj_ref[...].astype(jnp.bfloat16)
        acc_ref[...] = jnp.dot(lhs, s_ref[...],
                               preferred_element_type=jnp.float32)

        @pl.when(jnp.logical_and(p == 0, in_tail))
        def _():
            slot = i - (NB - CACHE_BLOCKS)
            cache_ref[pl.ds(pl.multiple_of(slot * BM, BM), BM), :] = lhs

    @pl.when(jnp.logical_and(p == 1, in_tail))
    def _():
        slot = i - (NB - CACHE_BLOCKS)
        lhs = cache_ref[pl.ds(pl.multiple_of(slot * BM, BM), BM), :]
        acc_ref[...] = jnp.dot(lhs, s_ref[...],
                               preferred_element_type=jnp.float32)

    acc = acc_ref[...]

    @pl.when(p == 0)
    def _():
        g = jnp.tanh(acc[:, :F] + b1_ref[...])
        z = jax.nn.sigmoid(acc[:, F:] + b2_ref[...])
        t_ref[pl.ds(i * BM, BM), :] = (g * z).astype(jnp.bfloat16)

    @pl.when(p == 1)
    def _():
        mu_ref[...] = acc[:, :F] + bmu_ref[...]
        ls_ref[...] = acc[:, F:] + bls_ref[...]


def kernel(x, adj, W1, b1, W2, b2, Wmu, bmu, Wls, bls):
    wc1 = jnp.concatenate([W1, W2], axis=1).astype(jnp.bfloat16)
    wc2 = jnp.concatenate([Wmu, Wls], axis=1).astype(jnp.bfloat16)
    b1r = b1.reshape(1, F)
    b2r = b2.reshape(1, F)
    bmur = bmu.reshape(1, F)
    blsr = bls.reshape(1, F)

    grid = (2, NB)

    def adj_idx(p, i):
        tail = jnp.logical_and(p == 1, i >= NB - CACHE_BLOCKS)
        return (jnp.where(tail, NB - CACHE_BLOCKS - 1, i), 0)

    adj_spec = pl.BlockSpec((BM, N), adj_idx)
    x_spec = pl.BlockSpec((N, F), lambda p, i: (0, 0))
    w_spec = pl.BlockSpec((F, 2 * F), lambda p, i: (0, 0))
    bias_spec = pl.BlockSpec((1, F), lambda p, i: (0, 0))
    out_spec = pl.BlockSpec((BM, F), lambda p, i: (jnp.where(p == 0, 0, i), 0))

    mu, ls = pl.pallas_call(
        _gcn_kernel,
        grid=grid,
        in_specs=[adj_spec, x_spec, w_spec, w_spec,
                  bias_spec, bias_spec, bias_spec, bias_spec],
        out_specs=[out_spec, out_spec],
        out_shape=[jax.ShapeDtypeStruct((N, F), jnp.float32),
                   jax.ShapeDtypeStruct((N, F), jnp.float32)],
        scratch_shapes=[
            pltpu.VMEM((N, 2 * F), jnp.bfloat16),
            pltpu.VMEM((N, F), jnp.bfloat16),
            pltpu.VMEM((CACHE_BLOCKS * BM, N), jnp.bfloat16),
            pltpu.VMEM((BM, 2 * F), jnp.float32),
        ],
        compiler_params=pltpu.CompilerParams(
            dimension_semantics=("arbitrary", "arbitrary"),
            vmem_limit_bytes=64 * 1024 * 1024,
        ),
    )(adj, x, wc1, wc2, b1r, b2r, bmur, blsr)
    return (mu, ls)


# speedup vs baseline: 1.0013x; 1.0013x over previous
import jax
import jax.numpy as jnp
from jax.experimental import pallas as pl
from jax.experimental.pallas import tpu as pltpu

N = 10000
F = 128
BM = 400
BH = BM // 2


def _gcn_kernel(adja_ref, adjb_ref, x_ref, w1_ref, w2_ref, b1_ref, b2_ref,
                bmu_ref, bls_ref, mu_ref, ls_ref,
                s_ref, t_ref):
    p = pl.program_id(0)
    i = pl.program_id(1)

    @pl.when(jnp.logical_and(p == 0, i == 0))
    def _():
        s_ref[...] = jnp.dot(
            x_ref[...].astype(jnp.bfloat16), w1_ref[...],
            preferred_element_type=jnp.float32).astype(jnp.bfloat16)

    @pl.when(jnp.logical_and(p == 1, i == 0))
    def _():
        s_ref[...] = jnp.dot(
            t_ref[...], w2_ref[...],
            preferred_element_type=jnp.float32).astype(jnp.bfloat16)

    acc = jnp.concatenate([
        jnp.dot(adja_ref[...].astype(jnp.bfloat16), s_ref[...],
                preferred_element_type=jnp.float32),
        jnp.dot(adjb_ref[...].astype(jnp.bfloat16), s_ref[...],
                preferred_element_type=jnp.float32)], axis=0)

    @pl.when(p == 0)
    def _():
        g = jnp.tanh(acc[:, :F] + b1_ref[...])
        z = jax.nn.sigmoid(acc[:, F:] + b2_ref[...])
        t_ref[pl.ds(i * BM, BM), :] = (g * z).astype(jnp.bfloat16)

    @pl.when(p == 1)
    def _():
        mu_ref[...] = acc[:, :F] + bmu_ref[...]
        ls_ref[...] = acc[:, F:] + bls_ref[...]


def kernel(x, adj, W1, b1, W2, b2, Wmu, bmu, Wls, bls):
    wc1 = jnp.concatenate([W1, W2], axis=1).astype(jnp.bfloat16)
    wc2 = jnp.concatenate([Wmu, Wls], axis=1).astype(jnp.bfloat16)
    b1r = b1.reshape(1, F)
    b2r = b2.reshape(1, F)
    bmur = bmu.reshape(1, F)
    blsr = bls.reshape(1, F)

    grid = (2, N // BM)
    adja_spec = pl.BlockSpec((BH, N), lambda p, i: (2 * i, 0))
    adjb_spec = pl.BlockSpec((BH, N), lambda p, i: (2 * i + 1, 0))
    x_spec = pl.BlockSpec((N, F), lambda p, i: (0, 0))
    w_spec = pl.BlockSpec((F, 2 * F), lambda p, i: (0, 0))
    bias_spec = pl.BlockSpec((1, F), lambda p, i: (0, 0))
    out_spec = pl.BlockSpec((BM, F), lambda p, i: (jnp.where(p == 0, 0, i), 0))

    mu, ls = pl.pallas_call(
        _gcn_kernel,
        grid=grid,
        in_specs=[adja_spec, adjb_spec, x_spec, w_spec, w_spec,
                  bias_spec, bias_spec, bias_spec, bias_spec],
        out_specs=[out_spec, out_spec],
        out_shape=[jax.ShapeDtypeStruct((N, F), jnp.float32),
                   jax.ShapeDtypeStruct((N, F), jnp.float32)],
        scratch_shapes=[pltpu.VMEM((N, 2 * F), jnp.bfloat16),
                        pltpu.VMEM((N, F), jnp.bfloat16)],
        compiler_params=pltpu.CompilerParams(
            dimension_semantics=("arbitrary", "arbitrary"),
            vmem_limit_bytes=64 * 1024 * 1024,
        ),
    )(adj, adj, x, wc1, wc2, b1r, b2r, bmur, blsr)
    return (mu, ls)


# speedup vs baseline: 1.0221x; 1.0207x over previous
import jax
import jax.numpy as jnp
from jax.experimental import pallas as pl
from jax.experimental.pallas import tpu as pltpu

N = 10000
F = 128
BM = 400


def _gcn_kernel(adj_ref, x_ref, w1_ref, w2_ref, b1_ref, b2_ref,
                bmu_ref, bls_ref, mu_ref, ls_ref,
                s_ref, t_ref):
    p = pl.program_id(0)
    i = pl.program_id(1)

    @pl.when(jnp.logical_and(p == 0, i == 0))
    def _():
        s_ref[...] = jnp.dot(
            x_ref[...].astype(jnp.bfloat16), w1_ref[...],
            preferred_element_type=jnp.float32).astype(jnp.bfloat16)

    @pl.when(jnp.logical_and(p == 1, i == 0))
    def _():
        s_ref[...] = jnp.dot(
            t_ref[...], w2_ref[...],
            preferred_element_type=jnp.float32).astype(jnp.bfloat16)

    acc = jnp.dot(adj_ref[...].astype(jnp.bfloat16), s_ref[...],
                  preferred_element_type=jnp.float32)

    @pl.when(p == 0)
    def _():
        g = jnp.tanh(acc[:, :F] + b1_ref[...])
        z = jax.nn.sigmoid(acc[:, F:] + b2_ref[...])
        t_ref[pl.ds(i * BM, BM), :] = (g * z).astype(jnp.bfloat16)

    @pl.when(p == 1)
    def _():
        mu_ref[...] = acc[:, :F] + bmu_ref[...]
        ls_ref[...] = acc[:, F:] + bls_ref[...]


def kernel(x, adj, W1, b1, W2, b2, Wmu, bmu, Wls, bls):
    wc1 = jnp.concatenate([W1, W2], axis=1).astype(jnp.bfloat16)
    wc2 = jnp.concatenate([Wmu, Wls], axis=1).astype(jnp.bfloat16)
    b1r = b1.reshape(1, F)
    b2r = b2.reshape(1, F)
    bmur = bmu.reshape(1, F)
    blsr = bls.reshape(1, F)

    grid = (2, N // BM)
    nb = N // BM
    adj_spec = pl.BlockSpec((BM, N), lambda p, i: (jnp.where(p == 0, i, nb - 1 - i), 0))
    x_spec = pl.BlockSpec((N, F), lambda p, i: (0, 0))
    w_spec = pl.BlockSpec((F, 2 * F), lambda p, i: (0, 0))
    bias_spec = pl.BlockSpec((1, F), lambda p, i: (0, 0))
    out_spec = pl.BlockSpec((BM, F), lambda p, i: (jnp.where(p == 0, nb - 1, nb - 1 - i), 0))

    mu, ls = pl.pallas_call(
        _gcn_kernel,
        grid=grid,
        in_specs=[adj_spec, x_spec, w_spec, w_spec,
                  bias_spec, bias_spec, bias_spec, bias_spec],
        out_specs=[out_spec, out_spec],
        out_shape=[jax.ShapeDtypeStruct((N, F), jnp.float32),
                   jax.ShapeDtypeStruct((N, F), jnp.float32)],
        scratch_shapes=[pltpu.VMEM((N, 2 * F), jnp.bfloat16),
                        pltpu.VMEM((N, F), jnp.bfloat16)],
        compiler_params=pltpu.CompilerParams(
            dimension_semantics=("arbitrary", "arbitrary"),
            vmem_limit_bytes=64 * 1024 * 1024,
        ),
    )(adj, x, wc1, wc2, b1r, b2r, bmur, blsr)
    return (mu, ls)
